# baseline (speedup 1.0000x reference)
import functools

import jax
import jax.numpy as jnp
from jax import lax
from jax.experimental import pallas as pl
from jax.experimental.pallas import tpu as pltpu
from jax.experimental.pallas import tpu_sc as plsc

N = 10000
E = 320000
H = 128
HC = H // 2
T = 4
TAU_INV = 0.5
VTH = 1.0

NTILES = 16
EPT = E // NTILES
CH = 64
EPT_PAD = 20480
NCH = EPT_PAD // CH
SB = 32
NSB = NCH // SB
D = 8
NP = 10240
RPT = NP // 16
ZR = 32


@functools.cache
def _get_prop_kernel():
    mesh = plsc.VectorSubcoreMesh(core_axis_name="c", subcore_axis_name="s")
    return pl.kernel(
        _prop_body_sc,
        mesh=mesh,
        compiler_params=pltpu.CompilerParams(use_tc_tiling_on_sc=False),
        out_type=jax.ShapeDtypeStruct((2, NP, HC), jnp.float32),
        scratch_types=[
            pltpu.VMEM((SB, CH), jnp.int32),
            pltpu.VMEM((SB, CH), jnp.int32),
            pltpu.VMEM((ZR, HC), jnp.float32),
            pltpu.VMEM_SHARED((NP, HC), jnp.float32),
        ] + [pltpu.VMEM((CH, HC), jnp.float32) for _ in range(D)]
          + [pltpu.SemaphoreType.DMA for _ in range(2 * D)],
    )


def _prop_body_sc(u_hbm, src_hbm, dst_hbm, out_hbm, sib, dib, zb, acc,
                  *bufs):
    rows = bufs[:D]
    gss = bufs[D:2 * D]
    sss = bufs[2 * D:]
    core = lax.axis_index("c")
    sid = lax.axis_index("s")

    def zb_body(i, _):
        zb[i // (HC // 16), pl.ds((i % (HC // 16)) * 16, 16)] = (
            jnp.zeros((16,), jnp.float32))
        return 0

    lax.fori_loop(0, ZR * (HC // 16), zb_body, 0)
    for b in range(RPT // ZR):
        pltpu.sync_copy(zb, acc.at[pl.ds(sid * RPT + b * ZR, ZR)])
    plsc.subcore_barrier()

    def sb_body(b, _):
        pltpu.sync_copy(src_hbm.at[core, sid, pl.ds(b * SB, SB)], sib)
        pltpu.sync_copy(dst_hbm.at[sid, pl.ds(b * SB, SB)], dib)
        for k in range(D):
            pltpu.async_copy(u_hbm.at[sib.at[k]], rows[k], gss[k])

        def chunk_body(q, _):
            j0 = D * q
            for k in range(D):
                j = j0 + k
                pltpu.make_async_copy(u_hbm.at[sib.at[j]], rows[k],
                                      gss[k]).wait()
                pltpu.async_copy(rows[k], acc.at[dib.at[j]], sss[k],
                                 add=True)
            for k in range(D):
                j = j0 + k
                pltpu.make_async_copy(rows[k], acc.at[dib.at[j]],
                                      sss[k]).wait()
                pltpu.async_copy(u_hbm.at[sib.at[j + D]], rows[k], gss[k])
            return 0

        lax.fori_loop(0, SB // D - 1, chunk_body, 0)
        for k in range(D):
            j = SB - D + k
            pltpu.make_async_copy(u_hbm.at[sib.at[j]], rows[k], gss[k]).wait()
            pltpu.sync_copy(rows[k], acc.at[dib.at[j]], add=True)
        return 0

    lax.fori_loop(0, NSB, sb_body, 0)
    plsc.subcore_barrier()
    pltpu.sync_copy(acc.at[pl.ds(sid * RPT, RPT)],
                    out_hbm.at[core, pl.ds(sid * RPT, RPT)])


_BM = 2000
_BD = 1280


def _mm2_body(x_ref, w1_ref, b1_ref, w2_ref, b2_ref, s_ref, o_ref):
    t = jnp.dot(x_ref[...], w1_ref[...],
                preferred_element_type=jnp.float32) + b1_ref[...]
    y = jnp.dot(t, w2_ref[...], preferred_element_type=jnp.float32) + b2_ref[...]
    o_ref[0] = s_ref[0] * y[:, :HC]
    o_ref[1] = s_ref[1] * y[:, HC:]


def _mm_scaled_body(x_ref, w_ref, b_ref, s_ref, o_ref):
    xv = jnp.concatenate([x_ref[0, 0], x_ref[0, 1]], axis=-1)
    y = jnp.dot(xv, w_ref[...], preferred_element_type=jnp.float32) + b_ref[...]
    o_ref[0, 0] = s_ref[0] * y[:, :HC]
    o_ref[0, 1] = s_ref[1] * y[:, HC:]


def _mm_plain_body(x_ref, w_ref, b_ref, o_ref):
    xv = jnp.concatenate([x_ref[0], x_ref[1]], axis=-1)
    o_ref[...] = (jnp.dot(xv, w_ref[...],
                          preferred_element_type=jnp.float32) + b_ref[...])


def _combine_body(p_ref, u_ref, s_ref, o_ref):
    o_ref[...] = s_ref[...] * (p_ref[...] + u_ref[...])


def _lif0_body(g_ref, o_ref):
    gv = g_ref[...]
    v = jnp.zeros_like(gv)
    for t in range(T):
        v = v + (gv - v) * TAU_INV
        s = (v >= VTH).astype(jnp.float32)
        o_ref[t] = s
        v = v * (1.0 - s)


def _lif1_body(g_ref, o_ref):
    v = jnp.zeros_like(g_ref[0])
    a = jnp.zeros_like(v)
    for t in range(T):
        v = v + (g_ref[t] - v) * TAU_INV
        s = (v >= VTH).astype(jnp.float32)
        a = a + s
        v = v * (1.0 - s)
    o_ref[...] = a * (1.0 / T)


def _deg_fin_body(p_ref, di_ref, d2_ref):
    d = p_ref[...] + 1.0
    di_ref[...] = lax.rsqrt(d)
    d2_ref[...] = 1.0 / d


def _row_spec(bm, w):
    return pl.BlockSpec((bm, w), lambda i: (i, 0))


def _split_spec(bm):
    return pl.BlockSpec((2, bm, HC), lambda i: (0, i, 0))


def _full_spec(shape):
    nd = len(shape)
    return pl.BlockSpec(shape, lambda i: (0,) * nd)


def _mm2(x, w1, b1, w2, b2, s):
    grid = (x.shape[0] // _BM,)
    return pl.pallas_call(
        _mm2_body,
        grid=grid,
        in_specs=[_row_spec(_BM, H), _full_spec((H, H)), _full_spec((1, H)),
                  _full_spec((H, H)), _full_spec((1, H)), _split_spec(_BM)],
        out_specs=_split_spec(_BM),
        out_shape=jax.ShapeDtypeStruct((2, x.shape[0], HC), jnp.float32),
    )(x, w1, b1, w2, b2, s)


def _mm_scaled(x, w, b, s):
    nsb = N // _BM
    grid = (T * nsb,)
    return pl.pallas_call(
        _mm_scaled_body,
        grid=grid,
        in_specs=[pl.BlockSpec((1, 2, _BM, HC),
                               lambda i: (i // nsb, 0, i % nsb, 0)),
                  _full_spec((H, H)), _full_spec((1, H)),
                  pl.BlockSpec((2, _BM, HC), lambda i: (0, i % nsb, 0))],
        out_specs=pl.BlockSpec((1, 2, _BM, HC),
                               lambda i: (i // nsb, 0, i % nsb, 0)),
        out_shape=jax.ShapeDtypeStruct((T, 2, N, HC), jnp.float32),
    )(x, w, b, s)


def _mm_plain(x, w, b):
    grid = (N // _BM,)
    return pl.pallas_call(
        _mm_plain_body,
        grid=grid,
        in_specs=[_split_spec(_BM), _full_spec((H, H)), _full_spec((1, H))],
        out_specs=_row_spec(_BM, H),
        out_shape=jax.ShapeDtypeStruct((N, w.shape[1]), jnp.float32),
    )(x, w, b)


def _combine(p, u, s):
    grid = (N // _BM,)
    return pl.pallas_call(
        _combine_body,
        grid=grid,
        in_specs=[_split_spec(_BM), _split_spec(_BM), _split_spec(_BM)],
        out_specs=_split_spec(_BM),
        out_shape=jax.ShapeDtypeStruct((2, N, HC), jnp.float32),
    )(p, u, s)


def _lif0(g):
    grid = (N // _BM,)
    return pl.pallas_call(
        _lif0_body,
        grid=grid,
        in_specs=[_split_spec(_BM)],
        out_specs=pl.BlockSpec((T, 2, _BM, HC), lambda i: (0, 0, i, 0)),
        out_shape=jax.ShapeDtypeStruct((T, 2, N, HC), jnp.float32),
    )(g)


def _lif1(g):
    grid = (N // _BM,)
    return pl.pallas_call(
        _lif1_body,
        grid=grid,
        in_specs=[pl.BlockSpec((T, 2, _BM, HC), lambda i: (0, 0, i, 0))],
        out_specs=_split_spec(_BM),
        out_shape=jax.ShapeDtypeStruct((2, N, HC), jnp.float32),
    )(g)


def _deg_fin(p):
    grid = (NP // _BD,)
    return pl.pallas_call(
        _deg_fin_body,
        grid=grid,
        in_specs=[_split_spec(_BD)],
        out_specs=[_split_spec(_BD), _split_spec(_BD)],
        out_shape=[jax.ShapeDtypeStruct((2, NP, HC), jnp.float32),
                   jax.ShapeDtypeStruct((2, NP, HC), jnp.float32)],
    )(p)


def kernel(x, edge_index, W_in, b_in, W_convs, b_convs, W_out, b_out):
    src = edge_index[0].astype(jnp.int32)
    dst = edge_index[1].astype(jnp.int32)
    pad = EPT_PAD - EPT
    src3 = jnp.concatenate(
        [src.reshape(NTILES, EPT),
         jnp.zeros((NTILES, pad), jnp.int32)], axis=1).reshape(NTILES, NCH, CH)
    src3c = jnp.stack([src3, src3 + N])
    dst3 = jnp.concatenate(
        [dst.reshape(NTILES, EPT),
         jnp.full((NTILES, pad), N, jnp.int32)], axis=1).reshape(NTILES, NCH, CH)

    prop_k = _get_prop_kernel()

    def prop(h_split):
        return prop_k(h_split.reshape(2 * N, HC), src3c, dst3)

    p = prop(jnp.ones((2, N, HC), jnp.float32))
    dinv_t, dinv2_t = _deg_fin(p)
    dinv, dinv2 = dinv_t, dinv2_t

    u0 = _mm2(x, W_in, b_in[None, :], W_convs[0], b_convs[0][None, :], dinv)
    u1 = _combine(prop(u0), u0, dinv2)
    g0 = _combine(prop(u1), u1, dinv)
    s0 = _lif0(g0)

    u_all = _mm_scaled(s0, W_convs[1], b_convs[1][None, :], dinv)
    outs = []
    for t in range(T):
        ut = u_all[t]
        u2 = _combine(prop(ut), ut, dinv2)
        outs.append(_combine(prop(u2), u2, dinv))
    a = _lif1(jnp.stack(outs))
    return _mm_plain(a, W_out, b_out[None, :])


# speedup vs baseline: 1.0229x; 1.0229x over previous
import functools

import jax
import jax.numpy as jnp
from jax import lax
from jax.experimental import pallas as pl
from jax.experimental.pallas import tpu as pltpu
from jax.experimental.pallas import tpu_sc as plsc

N = 10000
E = 320000
H = 128
HC = H // 2
T = 4
TAU_INV = 0.5
VTH = 1.0

NTILES = 16
EPT = E // NTILES
CH = 128
EPT_PAD = 20480
NCH = EPT_PAD // CH
SB = 32
NSB = NCH // SB
NP = 10240
RPT = NP // 16
ZR = 32


@functools.cache
def _get_prop_kernel():
    mesh = plsc.VectorSubcoreMesh(core_axis_name="c", subcore_axis_name="s")
    return pl.kernel(
        _prop_body_sc,
        mesh=mesh,
        compiler_params=pltpu.CompilerParams(use_tc_tiling_on_sc=False),
        out_type=jax.ShapeDtypeStruct((2, NP, HC), jnp.float32),
        scratch_types=[
            pltpu.VMEM((SB, CH), jnp.int32),
            pltpu.VMEM((SB, CH), jnp.int32),
            pltpu.VMEM((ZR, HC), jnp.float32),
            pltpu.VMEM_SHARED((NP, HC), jnp.float32),
            pltpu.VMEM((CH, HC), jnp.float32),
            pltpu.VMEM((CH, HC), jnp.float32),
            pltpu.VMEM((CH, HC), jnp.float32),
            pltpu.VMEM((CH, HC), jnp.float32),
            pltpu.SemaphoreType.DMA,
            pltpu.SemaphoreType.DMA,
            pltpu.SemaphoreType.DMA,
            pltpu.SemaphoreType.DMA,
            pltpu.SemaphoreType.DMA,
            pltpu.SemaphoreType.DMA,
            pltpu.SemaphoreType.DMA,
            pltpu.SemaphoreType.DMA,
        ],
    )


def _prop_body_sc(u_hbm, src_hbm, dst_hbm, out_hbm,
                  sib, dib, zb, acc, r0, r1, r2, r3,
                  gs0, gs1, gs2, gs3, ss0, ss1, ss2, ss3):
    core = lax.axis_index("c")
    sid = lax.axis_index("s")

    def zb_body(i, _):
        zb[i // (HC // 16), pl.ds((i % (HC // 16)) * 16, 16)] = (
            jnp.zeros((16,), jnp.float32))
        return 0

    lax.fori_loop(0, ZR * (HC // 16), zb_body, 0)
    for b in range(RPT // ZR):
        pltpu.sync_copy(zb, acc.at[pl.ds(sid * RPT + b * ZR, ZR)])
    plsc.subcore_barrier()

    def sb_body(b, _):
        pltpu.sync_copy(src_hbm.at[core, sid, pl.ds(b * SB, SB)], sib)
        pltpu.sync_copy(dst_hbm.at[sid, pl.ds(b * SB, SB)], dib)
        rows = (r0, r1, r2, r3)
        gss = (gs0, gs1, gs2, gs3)
        sss = (ss0, ss1, ss2, ss3)
        for k in range(4):
            pltpu.async_copy(u_hbm.at[sib.at[k]], rows[k], gss[k])

        def chunk_body(q, _):
            j0 = 4 * q
            for k in range(4):
                j = j0 + k
                pltpu.make_async_copy(u_hbm.at[sib.at[j]], rows[k],
                                      gss[k]).wait()
                pltpu.async_copy(rows[k], acc.at[dib.at[j]], sss[k],
                                 add=True)
            for k in range(4):
                j = j0 + k
                pltpu.make_async_copy(rows[k], acc.at[dib.at[j]],
                                      sss[k]).wait()
                pltpu.async_copy(u_hbm.at[sib.at[j + 4]], rows[k], gss[k])
            return 0

        lax.fori_loop(0, SB // 4 - 1, chunk_body, 0)
        for k in range(4):
            j = SB - 4 + k
            pltpu.make_async_copy(u_hbm.at[sib.at[j]], rows[k], gss[k]).wait()
            pltpu.sync_copy(rows[k], acc.at[dib.at[j]], add=True)
        return 0

    lax.fori_loop(0, NSB, sb_body, 0)
    plsc.subcore_barrier()
    pltpu.sync_copy(acc.at[pl.ds(sid * RPT, RPT)],
                    out_hbm.at[core, pl.ds(sid * RPT, RPT)])


_BM = 2000
_BD = 1280


def _mm2_body(x_ref, w1_ref, b1_ref, w2_ref, b2_ref, s_ref, o_ref):
    t = jnp.dot(x_ref[...], w1_ref[...],
                preferred_element_type=jnp.float32) + b1_ref[...]
    y = jnp.dot(t, w2_ref[...], preferred_element_type=jnp.float32) + b2_ref[...]
    o_ref[0] = s_ref[0] * y[:, :HC]
    o_ref[1] = s_ref[1] * y[:, HC:]


def _mm_scaled_body(x_ref, w_ref, b_ref, s_ref, o_ref):
    xv = jnp.concatenate([x_ref[0, 0], x_ref[0, 1]], axis=-1)
    y = jnp.dot(xv, w_ref[...], preferred_element_type=jnp.float32) + b_ref[...]
    o_ref[0, 0] = s_ref[0] * y[:, :HC]
    o_ref[0, 1] = s_ref[1] * y[:, HC:]


def _mm_plain_body(x_ref, w_ref, b_ref, o_ref):
    xv = jnp.concatenate([x_ref[0], x_ref[1]], axis=-1)
    o_ref[...] = (jnp.dot(xv, w_ref[...],
                          preferred_element_type=jnp.float32) + b_ref[...])


def _combine_body(p_ref, u_ref, s_ref, o_ref):
    o_ref[...] = s_ref[...] * (p_ref[...] + u_ref[...])


def _lif0_body(g_ref, o_ref):
    gv = g_ref[...]
    v = jnp.zeros_like(gv)
    for t in range(T):
        v = v + (gv - v) * TAU_INV
        s = (v >= VTH).astype(jnp.float32)
        o_ref[t] = s
        v = v * (1.0 - s)


def _lif1_body(g_ref, o_ref):
    v = jnp.zeros_like(g_ref[0])
    a = jnp.zeros_like(v)
    for t in range(T):
        v = v + (g_ref[t] - v) * TAU_INV
        s = (v >= VTH).astype(jnp.float32)
        a = a + s
        v = v * (1.0 - s)
    o_ref[...] = a * (1.0 / T)


def _deg_fin_body(p_ref, di_ref, d2_ref):
    d = p_ref[...] + 1.0
    di_ref[...] = lax.rsqrt(d)
    d2_ref[...] = 1.0 / d


def _row_spec(bm, w):
    return pl.BlockSpec((bm, w), lambda i: (i, 0))


def _split_spec(bm):
    return pl.BlockSpec((2, bm, HC), lambda i: (0, i, 0))


def _full_spec(shape):
    nd = len(shape)
    return pl.BlockSpec(shape, lambda i: (0,) * nd)


def _mm2(x, w1, b1, w2, b2, s):
    grid = (x.shape[0] // _BM,)
    return pl.pallas_call(
        _mm2_body,
        grid=grid,
        in_specs=[_row_spec(_BM, H), _full_spec((H, H)), _full_spec((1, H)),
                  _full_spec((H, H)), _full_spec((1, H)), _split_spec(_BM)],
        out_specs=_split_spec(_BM),
        out_shape=jax.ShapeDtypeStruct((2, x.shape[0], HC), jnp.float32),
    )(x, w1, b1, w2, b2, s)


def _mm_scaled(x, w, b, s):
    nsb = N // _BM
    grid = (T * nsb,)
    return pl.pallas_call(
        _mm_scaled_body,
        grid=grid,
        in_specs=[pl.BlockSpec((1, 2, _BM, HC),
                               lambda i: (i // nsb, 0, i % nsb, 0)),
                  _full_spec((H, H)), _full_spec((1, H)),
                  pl.BlockSpec((2, _BM, HC), lambda i: (0, i % nsb, 0))],
        out_specs=pl.BlockSpec((1, 2, _BM, HC),
                               lambda i: (i // nsb, 0, i % nsb, 0)),
        out_shape=jax.ShapeDtypeStruct((T, 2, N, HC), jnp.float32),
    )(x, w, b, s)


def _mm_plain(x, w, b):
    grid = (N // _BM,)
    return pl.pallas_call(
        _mm_plain_body,
        grid=grid,
        in_specs=[_split_spec(_BM), _full_spec((H, H)), _full_spec((1, H))],
        out_specs=_row_spec(_BM, H),
        out_shape=jax.ShapeDtypeStruct((N, w.shape[1]), jnp.float32),
    )(x, w, b)


def _combine(p, u, s):
    grid = (N // _BM,)
    return pl.pallas_call(
        _combine_body,
        grid=grid,
        in_specs=[_split_spec(_BM), _split_spec(_BM), _split_spec(_BM)],
        out_specs=_split_spec(_BM),
        out_shape=jax.ShapeDtypeStruct((2, N, HC), jnp.float32),
    )(p, u, s)


def _lif0(g):
    grid = (N // _BM,)
    return pl.pallas_call(
        _lif0_body,
        grid=grid,
        in_specs=[_split_spec(_BM)],
        out_specs=pl.BlockSpec((T, 2, _BM, HC), lambda i: (0, 0, i, 0)),
        out_shape=jax.ShapeDtypeStruct((T, 2, N, HC), jnp.float32),
    )(g)


def _lif1(g):
    grid = (N // _BM,)
    return pl.pallas_call(
        _lif1_body,
        grid=grid,
        in_specs=[pl.BlockSpec((T, 2, _BM, HC), lambda i: (0, 0, i, 0))],
        out_specs=_split_spec(_BM),
        out_shape=jax.ShapeDtypeStruct((2, N, HC), jnp.float32),
    )(g)


def _deg_fin(p):
    grid = (NP // _BD,)
    return pl.pallas_call(
        _deg_fin_body,
        grid=grid,
        in_specs=[_split_spec(_BD)],
        out_specs=[_split_spec(_BD), _split_spec(_BD)],
        out_shape=[jax.ShapeDtypeStruct((2, NP, HC), jnp.float32),
                   jax.ShapeDtypeStruct((2, NP, HC), jnp.float32)],
    )(p)


def kernel(x, edge_index, W_in, b_in, W_convs, b_convs, W_out, b_out):
    src = edge_index[0].astype(jnp.int32)
    dst = edge_index[1].astype(jnp.int32)
    pad = EPT_PAD - EPT
    src3 = jnp.concatenate(
        [src.reshape(NTILES, EPT),
         jnp.zeros((NTILES, pad), jnp.int32)], axis=1).reshape(NTILES, NCH, CH)
    src3c = jnp.stack([src3, src3 + N])
    dst3 = jnp.concatenate(
        [dst.reshape(NTILES, EPT),
         jnp.full((NTILES, pad), N, jnp.int32)], axis=1).reshape(NTILES, NCH, CH)

    prop_k = _get_prop_kernel()

    def prop(h_split):
        return prop_k(h_split.reshape(2 * N, HC), src3c, dst3)

    p = prop(jnp.ones((2, N, HC), jnp.float32))
    dinv_t, dinv2_t = _deg_fin(p)
    dinv, dinv2 = dinv_t, dinv2_t

    u0 = _mm2(x, W_in, b_in[None, :], W_convs[0], b_convs[0][None, :], dinv)
    u1 = _combine(prop(u0), u0, dinv2)
    g0 = _combine(prop(u1), u1, dinv)
    s0 = _lif0(g0)

    u_all = _mm_scaled(s0, W_convs[1], b_convs[1][None, :], dinv)
    outs = []
    for t in range(T):
        ut = u_all[t]
        u2 = _combine(prop(ut), ut, dinv2)
        outs.append(_combine(prop(u2), u2, dinv))
    a = _lif1(jnp.stack(outs))
    return _mm_plain(a, W_out, b_out[None, :])
